# baseline (speedup 1.0000x reference)
import functools

import jax
import jax.numpy as jnp
from jax import lax
from jax.experimental import pallas as pl
from jax.experimental.pallas import tpu as pltpu
from jax.experimental.pallas import tpu_sc as plsc

B = 4096
S = 200
P = 201
D = 64
NW = 32
COLS_PER_W = B // NW
NG = 3
NT = 2
PC_ROWS = 208

_mesh = plsc.VectorSubcoreMesh(core_axis_name="c", subcore_axis_name="s")


@functools.partial(
    pl.kernel,
    mesh=_mesh,
    out_type=jax.ShapeDtypeStruct((P, D, B), jnp.float32),
    scratch_types=[
        pltpu.VMEM((S, COLS_PER_W), jnp.int32),
        pltpu.VMEM((PC_ROWS, 128), jnp.float32),
        [pltpu.VMEM((COLS_PER_W, 128), jnp.float32) for _ in range(NG)],
        [pltpu.VMEM((D, COLS_PER_W), jnp.float32) for _ in range(NT)],
        [pltpu.SemaphoreType.DMA for _ in range(NG)],
        [pltpu.SemaphoreType.DMA for _ in range(NT)],
    ],
    compiler_params=pltpu.CompilerParams(use_tc_tiling_on_sc=True,
                                         needs_layout_passes=False,
                                         disable_bounds_checks=True),
)
def _sc_embed(idsT_hbm, table_hbm, poscls_hbm, out_hbm,
              ids_v, pc_v, gbufs, tbufs, gsems, ssems):
    wid = lax.axis_index("s") * 2 + lax.axis_index("c")
    base = wid * COLS_PER_W

    pltpu.sync_copy(idsT_hbm.at[:, pl.ds(base, COLS_PER_W)], ids_v)
    pltpu.sync_copy(poscls_hbm, pc_v)

    lanes = jnp.arange(16, dtype=jnp.int32)
    dvecs = [lanes + 16 * dc for dc in range(D // 16)]

    def fire_gather(s, k):
        pltpu.async_copy(table_hbm.at[ids_v.at[s]], gbufs[k], gsems[k])

    def wait_gather(s, k):
        pltpu.make_async_copy(table_hbm.at[ids_v.at[s]], gbufs[k],
                              gsems[k]).wait()

    def wait_store(kt):
        pltpu.make_async_copy(tbufs[kt],
                              out_hbm.at[0, :, pl.ds(base, COLS_PER_W)],
                              ssems[kt]).wait()

    rowvs = [lanes + 16 * j0 for j0 in range(COLS_PER_W // 16)]

    def txp_add(s, k, kt):
        g, t = gbufs[k], tbufs[kt]
        srow = jnp.full((16,), s, jnp.int32)

        @plsc.parallel_loop(0, 16, step=1, unroll=8)
        def _(r):
            diag = (lanes + r) & 15
            for d0 in range(0, D, 16):
                dvec = diag + d0
                pv = plsc.load_gather(pc_v, [srow, dvec])
                for j0 in range(COLS_PER_W // 16):
                    v = plsc.load_gather(g, [rowvs[j0], dvec])
                    plsc.store_scatter(t, [dvec, rowvs[j0]], v + pv)

    def finish_pos(s, k, kt, first):
        wait_gather(s, k)
        if not first:
            wait_store(kt)
        txp_add(s, k, kt)
        pltpu.async_copy(tbufs[kt],
                         out_hbm.at[s, :, pl.ds(base, COLS_PER_W)],
                         ssems[kt])

    fire_gather(0, 0)
    fire_gather(1, 1)
    finish_pos(0, 0, 0, True)
    fire_gather(2, 2)
    finish_pos(1, 1, 1, True)
    fire_gather(3, 0)

    def body(i, carry):
        for o in range(6):
            s = 6 * i + 2 + o
            finish_pos(s, (2 + o) % NG, o % NT, False)

            @pl.when(s + 2 < S)
            def _():
                fire_gather(s + 2, (4 + o) % NG)

        return carry

    lax.fori_loop(0, (S - 2) // 6, body, 0)

    wait_store(0)
    t0 = tbufs[0]
    r200 = jnp.full((16,), 200, jnp.int32)
    r201 = jnp.full((16,), 201, jnp.int32)

    @plsc.parallel_loop(0, D, step=1, unroll=2)
    def _(d):
        dcol = jnp.full((16,), d, jnp.int32)
        v = plsc.load_gather(pc_v, [r201, dcol]) + plsc.load_gather(
            pc_v, [r200, dcol])
        for jv in range(COLS_PER_W // 16):
            t0[d, pl.ds(16 * jv, 16)] = v

    pltpu.async_copy(t0, out_hbm.at[S, :, pl.ds(base, COLS_PER_W)], ssems[0])

    wait_store(0)
    wait_store(1)


@jax.jit
def kernel(input_ids, token_embeddings_weight, position_embeddings,
           cls_embedding):
    idsT = input_ids.T
    table_p = jnp.pad(token_embeddings_weight, ((0, 0), (0, 128 - D)))
    poscls = jnp.pad(
        jnp.concatenate([position_embeddings, cls_embedding[None, :]], 0),
        ((0, PC_ROWS - P - 1), (0, 128 - D)))
    y = _sc_embed(idsT, table_p, poscls)
    return jnp.transpose(y, (2, 0, 1))


# speedup vs baseline: 1.0639x; 1.0639x over previous
import functools

import jax
import jax.numpy as jnp
from jax import lax
from jax.experimental import pallas as pl
from jax.experimental.pallas import tpu as pltpu
from jax.experimental.pallas import tpu_sc as plsc

B = 4096
S = 200
P = 201
D = 64
NW = 32
COLS_PER_W = B // NW
NG = 3
NT = 2
PC_ROWS = 208

_mesh = plsc.VectorSubcoreMesh(core_axis_name="c", subcore_axis_name="s")


@functools.partial(
    pl.kernel,
    mesh=_mesh,
    out_type=jax.ShapeDtypeStruct((P, D, B), jnp.float32),
    scratch_types=[
        pltpu.VMEM((S, COLS_PER_W), jnp.int32),
        pltpu.VMEM((PC_ROWS, 128), jnp.float32),
        [pltpu.VMEM((COLS_PER_W, 128), jnp.float32) for _ in range(NG)],
        [pltpu.VMEM((D, COLS_PER_W), jnp.float32) for _ in range(NT)],
        [pltpu.SemaphoreType.DMA for _ in range(NG)],
        [pltpu.SemaphoreType.DMA for _ in range(NT)],
    ],
    compiler_params=pltpu.CompilerParams(use_tc_tiling_on_sc=True,
                                         needs_layout_passes=False,
                                         disable_bounds_checks=True),
)
def _sc_embed(idsT_hbm, table_hbm, poscls_hbm, out_hbm,
              ids_v, pc_v, gbufs, tbufs, gsems, ssems):
    wid = lax.axis_index("s") * 2 + lax.axis_index("c")
    base = wid * COLS_PER_W

    pltpu.sync_copy(idsT_hbm.at[:, pl.ds(base, COLS_PER_W)], ids_v)
    pltpu.sync_copy(poscls_hbm, pc_v)

    lanes = jnp.arange(16, dtype=jnp.int32)
    dvecs = [lanes + 16 * dc for dc in range(D // 16)]

    def fire_gather(s, k):
        pltpu.async_copy(table_hbm.at[ids_v.at[s]], gbufs[k], gsems[k])

    def wait_gather(s, k):
        pltpu.make_async_copy(table_hbm.at[ids_v.at[s]], gbufs[k],
                              gsems[k]).wait()

    def wait_store(kt):
        pltpu.make_async_copy(tbufs[kt],
                              out_hbm.at[0, :, pl.ds(base, COLS_PER_W)],
                              ssems[kt]).wait()

    rowvs = [lanes + 16 * j0 for j0 in range(COLS_PER_W // 16)]

    def txp_add(s, k, kt):
        g, t = gbufs[k], tbufs[kt]
        srow = jnp.full((16,), s, jnp.int32)

        @plsc.parallel_loop(0, 16, step=1, unroll=4)
        def _(r):
            diag = (lanes + r) & 15
            for d0 in range(0, D, 16):
                dvec = diag + d0
                pv = plsc.load_gather(pc_v, [srow, dvec])
                for j0 in range(COLS_PER_W // 16):
                    v = plsc.load_gather(g, [rowvs[j0], dvec])
                    plsc.store_scatter(t, [dvec, rowvs[j0]], v + pv)

    def finish_pos(s, k, kt, first):
        wait_gather(s, k)
        if not first:
            wait_store(kt)
        txp_add(s, k, kt)
        pltpu.async_copy(tbufs[kt],
                         out_hbm.at[s, :, pl.ds(base, COLS_PER_W)],
                         ssems[kt])

    fire_gather(0, 0)
    fire_gather(1, 1)
    finish_pos(0, 0, 0, True)
    fire_gather(2, 2)
    finish_pos(1, 1, 1, True)
    fire_gather(3, 0)

    def body(i, carry):
        for o in range(6):
            s = 6 * i + 2 + o
            finish_pos(s, (2 + o) % NG, o % NT, False)

            @pl.when(s + 2 < S)
            def _():
                fire_gather(s + 2, (4 + o) % NG)

        return carry

    lax.fori_loop(0, (S - 2) // 6, body, 0)

    wait_store(0)
    t0 = tbufs[0]
    r200 = jnp.full((16,), 200, jnp.int32)
    r201 = jnp.full((16,), 201, jnp.int32)

    @plsc.parallel_loop(0, D, step=1, unroll=2)
    def _(d):
        dcol = jnp.full((16,), d, jnp.int32)
        v = plsc.load_gather(pc_v, [r201, dcol]) + plsc.load_gather(
            pc_v, [r200, dcol])
        for jv in range(COLS_PER_W // 16):
            t0[d, pl.ds(16 * jv, 16)] = v

    pltpu.async_copy(t0, out_hbm.at[S, :, pl.ds(base, COLS_PER_W)], ssems[0])

    wait_store(0)
    wait_store(1)


@jax.jit
def kernel(input_ids, token_embeddings_weight, position_embeddings,
           cls_embedding):
    idsT = input_ids.T
    table_p = jnp.pad(token_embeddings_weight, ((0, 0), (0, 128 - D)))
    poscls = jnp.pad(
        jnp.concatenate([position_embeddings, cls_embedding[None, :]], 0),
        ((0, PC_ROWS - P - 1), (0, 128 - D)))
    y = _sc_embed(idsT, table_p, poscls)
    return jnp.transpose(y, (2, 0, 1))
